# Initial kernel scaffold; baseline (speedup 1.0000x reference)
import jax
import jax.numpy as jnp
from jax.experimental import pallas as pl


def kernel(x):
    raise NotImplementedError("write your pallas kernel here")


# speedup vs baseline: 4.5133x; 4.5133x over previous
import jax
import jax.numpy as jnp
from jax.experimental import pallas as pl
from jax.experimental.pallas import tpu as pltpu

_BR = 256
_BC = 2048


def _inclusive_scan(y):
    rows, cols = y.shape
    d = 1
    while d < cols:
        shifted = jnp.concatenate(
            [jnp.zeros((rows, d), y.dtype), y[:, :-d]], axis=1
        )
        y = y + shifted
        d *= 2
    return y


def _excl_cumsum_kernel(x_ref, o_ref, carry_ref):
    j = pl.program_id(1)

    @pl.when(j == 0)
    def _():
        carry_ref[...] = jnp.zeros_like(carry_ref)

    xb = x_ref[...]
    inc = _inclusive_scan(xb)
    carry = carry_ref[...]
    o_ref[...] = carry + (inc - xb)
    carry_ref[...] = carry + inc[:, -1:]


def kernel(x):
    B, N = x.shape
    grid = (B // _BR, N // _BC)
    return pl.pallas_call(
        _excl_cumsum_kernel,
        grid=grid,
        in_specs=[pl.BlockSpec((_BR, _BC), lambda i, j: (i, j))],
        out_specs=pl.BlockSpec((_BR, _BC), lambda i, j: (i, j)),
        out_shape=jax.ShapeDtypeStruct((B, N), x.dtype),
        scratch_shapes=[pltpu.VMEM((_BR, 1), jnp.float32)],
        compiler_params=pltpu.CompilerParams(
            dimension_semantics=("parallel", "arbitrary"),
        ),
    )(x)


# speedup vs baseline: 14.0148x; 3.1052x over previous
import jax
import jax.numpy as jnp
from jax.experimental import pallas as pl
from jax.experimental.pallas import tpu as pltpu

_BR = 4096
_BC = 512


def _excl_cumsum_kernel(x_ref, o_ref, carry_ref, t_ref):
    j = pl.program_id(1)

    @pl.when(j == 0)
    def _():
        carry_ref[...] = jnp.zeros_like(carry_ref)
        r = jax.lax.broadcasted_iota(jnp.int32, (_BC, _BC), 0)
        c = jax.lax.broadcasted_iota(jnp.int32, (_BC, _BC), 1)
        t_ref[...] = (r < c).astype(t_ref.dtype)

    xb = x_ref[...]
    excl = jax.lax.dot_general(
        xb, t_ref[...], (((1,), (0,)), ((), ())),
        preferred_element_type=jnp.float32,
    )
    carry = carry_ref[...]
    o_ref[...] = carry + excl
    carry_ref[...] = carry + excl[:, -1:] + xb[:, -1:]


def kernel(x):
    B, N = x.shape
    grid = (B // _BR, N // _BC)
    return pl.pallas_call(
        _excl_cumsum_kernel,
        grid=grid,
        in_specs=[pl.BlockSpec((_BR, _BC), lambda i, j: (i, j))],
        out_specs=pl.BlockSpec((_BR, _BC), lambda i, j: (i, j)),
        out_shape=jax.ShapeDtypeStruct((B, N), x.dtype),
        scratch_shapes=[
            pltpu.VMEM((_BR, 1), jnp.float32),
            pltpu.VMEM((_BC, _BC), jnp.float32),
        ],
        compiler_params=pltpu.CompilerParams(
            dimension_semantics=("parallel", "arbitrary"),
        ),
    )(x)
